# Initial kernel scaffold; baseline (speedup 1.0000x reference)
import jax
import jax.numpy as jnp
from jax.experimental import pallas as pl


def kernel(x, importance_scores):
    raise NotImplementedError("write your pallas kernel here")


# speedup vs baseline: 1.0962x; 1.0962x over previous
import jax
import jax.numpy as jnp
from jax.experimental import pallas as pl
from jax.experimental.pallas import tpu as pltpu

_NP = 3136
_NPAD = 3200
_PATCH = 4
_NUCLEUS_P = 0.2
_TEMP = 10.0
_GRAD_SCALE = 0.1
_MIN_K = 156
_MAX_K = 1568
_BISECT_ITERS = 48
_CB = 48


def _mask_kernel(s_ref, o_ref):
    v = s_ref[...]
    valid = v >= 0.0
    vz = jnp.where(valid, v, 0.0)
    tot = jnp.sum(vz, axis=-1, keepdims=True)
    denom = tot + 1e-8

    def body1(_, carry):
        lo, hi = carry
        mid = (lo + hi) * 0.5
        s = jnp.sum(jnp.where(v > mid, vz, 0.0), axis=-1, keepdims=True)
        qt = (s / denom) < _NUCLEUS_P
        return jnp.where(qt, lo, mid), jnp.where(qt, mid, hi)

    lo0 = jnp.zeros_like(tot)
    hi0 = jnp.ones_like(tot)
    _, w = jax.lax.fori_loop(0, _BISECT_ITERS, body1, (lo0, hi0))

    s_w = jnp.sum(jnp.where(v > w, vz, 0.0), axis=-1, keepdims=True)
    c_w = jnp.sum(jnp.where(v > w, 1.0, 0.0), axis=-1, keepdims=True)
    m_w = jnp.sum(jnp.where(v == w, 1.0, 0.0), axis=-1, keepdims=True)
    r = jnp.ceil((_NUCLEUS_P * denom - s_w) / jnp.maximum(w, 1e-30)) - 1.0
    r = jnp.clip(r, 0.0, jnp.maximum(m_w - 1.0, 0.0))
    counts = c_w + r + 1.0
    k = jnp.clip(counts, float(_MIN_K), float(_MAX_K))

    def body2(_, carry):
        lo, hi = carry
        mid = (lo + hi) * 0.5
        cnt = jnp.sum(jnp.where(v >= mid, 1.0, 0.0), axis=-1, keepdims=True)
        rt = cnt >= k
        return jnp.where(rt, mid, lo), jnp.where(rt, hi, mid)

    thr, _ = jax.lax.fori_loop(0, _BISECT_ITERS, body2, (lo0, hi0))

    g = v * _GRAD_SCALE + v * (1.0 - _GRAD_SCALE)
    soft = jax.nn.sigmoid(_TEMP * (g - thr))
    hard = (soft > 0.5).astype(jnp.float32)
    o_ref[...] = (hard - soft) + soft


def _expand_kernel(m_ref, p_ref, o_ref):
    m = m_ref[0]
    p = p_ref[...]
    up1 = jax.lax.dot_general(p, m, (((1,), (0,)), ((), ())),
                              preferred_element_type=jnp.float32)
    up2 = jax.lax.dot_general(up1, p, (((1,), (1,)), ((), ())),
                              preferred_element_type=jnp.float32)
    o_ref[0] = jnp.broadcast_to(up2[None], (o_ref.shape[1],) + up2.shape)


def kernel(x, importance_scores):
    B, C, H, W = x.shape
    hp, wp = H // _PATCH, W // _PATCH

    spad = jnp.pad(importance_scores, ((0, 0), (0, _NPAD - _NP)),
                   constant_values=-1.0)
    maskf = pl.pallas_call(
        _mask_kernel,
        out_shape=jax.ShapeDtypeStruct((B, _NPAD), jnp.float32),
    )(spad)
    m3 = maskf[:, :_NP].reshape(B, hp, wp)

    expand = (jnp.arange(H)[:, None] // _PATCH ==
              jnp.arange(hp)[None, :]).astype(jnp.float32)

    out = pl.pallas_call(
        _expand_kernel,
        grid=(B, C // _CB),
        in_specs=[
            pl.BlockSpec((1, hp, wp), lambda b, c: (b, 0, 0)),
            pl.BlockSpec((H, hp), lambda b, c: (0, 0)),
        ],
        out_specs=pl.BlockSpec((1, _CB, H, W), lambda b, c: (b, c, 0, 0)),
        out_shape=jax.ShapeDtypeStruct((B, C, H, W), jnp.float32),
        compiler_params=pltpu.CompilerParams(
            dimension_semantics=("parallel", "parallel")),
    )(m3, expand)
    return out[:, None]


# speedup vs baseline: 1.2372x; 1.1286x over previous
import jax
import jax.numpy as jnp
from jax.experimental import pallas as pl
from jax.experimental.pallas import tpu as pltpu

_NP = 3136
_NPAD = 3200
_PATCH = 4
_NUCLEUS_P = 0.2
_TEMP = 10.0
_GRAD_SCALE = 0.1
_MIN_K = 156
_MAX_K = 1568
_ITERS1 = 14
_ITERS2 = 24


def _quad_search(pred, lo0, hi0, iters):
    def body(_, carry):
        lo, hi = carry
        m2 = (lo + hi) * 0.5
        m1 = (lo + m2) * 0.5
        m3 = (m2 + hi) * 0.5
        q1, q2, q3 = pred(m1), pred(m2), pred(m3)
        hi2 = jnp.where(q1, m1, jnp.where(q2, m2, jnp.where(q3, m3, hi)))
        lo2 = jnp.where(q1, lo, jnp.where(q2, m1, jnp.where(q3, m2, m3)))
        return lo2, hi2
    return jax.lax.fori_loop(0, iters, body, (lo0, hi0))


def _fused_kernel(s_ref, m3_ref, p_ref, o_ref, t_ref, sem):
    B, C = o_ref.shape[0], o_ref.shape[1]
    v = s_ref[...]
    vz = jnp.where(v >= 0.0, v, 0.0)
    tot = jnp.sum(vz, axis=-1, keepdims=True)
    denom = tot + 1e-8

    lo0 = jnp.zeros_like(tot)
    hi0 = jnp.ones_like(tot)

    def q_pred(t):
        s = jnp.sum(jnp.where(v > t, vz, 0.0), axis=-1, keepdims=True)
        return (s / denom) < _NUCLEUS_P

    _, w = _quad_search(q_pred, lo0, hi0, _ITERS1)

    s_w = jnp.sum(jnp.where(v > w, vz, 0.0), axis=-1, keepdims=True)
    c_w = jnp.sum(jnp.where(v > w, 1.0, 0.0), axis=-1, keepdims=True)
    m_w = jnp.sum(jnp.where(v == w, 1.0, 0.0), axis=-1, keepdims=True)
    r = jnp.ceil((_NUCLEUS_P * denom - s_w) / jnp.maximum(w, 1e-30)) - 1.0
    r = jnp.clip(r, 0.0, jnp.maximum(m_w - 1.0, 0.0))
    k = jnp.clip(c_w + r + 1.0, float(_MIN_K), float(_MAX_K))

    def r_pred(t):
        cnt = jnp.sum(jnp.where(v >= t, 1.0, 0.0), axis=-1, keepdims=True)
        return cnt < k

    lo_t, _ = _quad_search(r_pred, lo0, hi0, _ITERS2)

    p = p_ref[...]
    for b in range(B):
        thr = lo_t[b:b + 1, :]
        s2 = m3_ref[b]
        g = s2 * _GRAD_SCALE + s2 * (1.0 - _GRAD_SCALE)
        soft = jax.nn.sigmoid(_TEMP * (g - thr))
        hard = (soft > 0.5).astype(jnp.float32)
        m = (hard - soft) + soft
        up1 = jax.lax.dot_general(p, m, (((1,), (0,)), ((), ())),
                                  preferred_element_type=jnp.float32)
        up2 = jax.lax.dot_general(up1, p, (((1,), (1,)), ((), ())),
                                  preferred_element_type=jnp.float32)
        t_ref[b] = up2
        def start(c, _, b=b):
            pltpu.make_async_copy(t_ref.at[b], o_ref.at[b, c], sem).start()
            return 0
        jax.lax.fori_loop(0, C, start, 0)

    def wait(i, _):
        pltpu.make_async_copy(t_ref.at[0], o_ref.at[0, 0], sem).wait()
        return 0
    jax.lax.fori_loop(0, B * C, wait, 0)


def kernel(x, importance_scores):
    B, C, H, W = x.shape
    hp, wp = H // _PATCH, W // _PATCH

    spad = jnp.pad(importance_scores, ((0, 0), (0, _NPAD - _NP)),
                   constant_values=-1.0)
    s3 = importance_scores.reshape(B, hp, wp)
    expand = (jnp.arange(H)[:, None] // _PATCH ==
              jnp.arange(hp)[None, :]).astype(jnp.float32)

    out = pl.pallas_call(
        _fused_kernel,
        in_specs=[
            pl.BlockSpec(memory_space=pltpu.MemorySpace.VMEM),
            pl.BlockSpec(memory_space=pltpu.MemorySpace.VMEM),
            pl.BlockSpec(memory_space=pltpu.MemorySpace.VMEM),
        ],
        out_specs=pl.BlockSpec(memory_space=pltpu.MemorySpace.HBM),
        out_shape=jax.ShapeDtypeStruct((B, C, H, W), jnp.float32),
        scratch_shapes=[
            pltpu.VMEM((B, H, W), jnp.float32),
            pltpu.SemaphoreType.DMA,
        ],
    )(spad, s3, expand)
    return out[:, None]
